# baseline (speedup 1.0000x reference)
import functools

import jax
import jax.numpy as jnp
from jax import lax
from jax.experimental import pallas as pl
from jax.experimental.pallas import tpu as pltpu
from jax.experimental.pallas import tpu_sc as plsc

_H = 4096
_W = 4096
_HW = _H * _W
_P = 500000
_THETA = 0.5

_NC = 2
_NS = 16
_NW = _NC * _NS

_PPAD = 500224
_G = _PPAD // _NW
_NV = _G // 16
_NIDX = 2 * _G
_CHUNK = 128
_NCH_MAX = (_NIDX + _CHUNK - 1) // _CHUNK
_INFLIGHT = 32


def _copy_body(x_ref, o_ref):
    o_ref[...] = x_ref[...]


_tc_copy = pl.pallas_call(
    _copy_body,
    out_shape=jax.ShapeDtypeStruct((_H, _W), jnp.float32),
    grid=(32,),
    in_specs=[pl.BlockSpec((_H // 32, _W), lambda i: (i, 0))],
    out_specs=pl.BlockSpec((_H // 32, _W), lambda i: (i, 0)),
)


def _sc_body(out_hbm, r0_h, c0_h, r1_h, c1_h, b_h, d_h,
             r0_v, c0_v, r1_v, c1_v, b_v, d_v, idx2d, zeros_v, sem):
    wid = lax.axis_index("s") * _NC + lax.axis_index("c")
    base = wid * _G

    pltpu.sync_copy(r0_h.at[pl.ds(base, _G)], r0_v)
    pltpu.sync_copy(c0_h.at[pl.ds(base, _G)], c0_v)
    pltpu.sync_copy(r1_h.at[pl.ds(base, _G)], r1_v)
    pltpu.sync_copy(c1_h.at[pl.ds(base, _G)], c1_v)
    pltpu.sync_copy(b_h.at[pl.ds(base, _G)], b_v)
    pltpu.sync_copy(d_h.at[pl.ds(base, _G)], d_v)

    for j in range(_CHUNK // 16):
        zeros_v[pl.ds(j * 16, 16)] = jnp.zeros((16,), jnp.float32)

    for j in range(_CHUNK // 16):
        plsc.store_scatter(
            idx2d,
            [jnp.full((16,), _NCH_MAX - 1, jnp.int32),
             j * 16 + lax.iota(jnp.int32, 16)],
            jnp.full((16,), -1, jnp.int32))

    def gen_body(i, _):
        sl = pl.ds(i * 16, 16)
        m = jnp.abs(d_v[sl] - b_v[sl]) <= _THETA
        f0 = jnp.where(m, r0_v[sl] * _W + c0_v[sl], -1)
        f1 = jnp.where(m, r1_v[sl] * _W + c1_v[sl], -1)
        rowv = jnp.full((16,), i >> 2, jnp.int32)
        colv = (i & 3) * 32 + lax.iota(jnp.int32, 16)
        plsc.store_scatter(idx2d, [rowv, colv], f0)
        plsc.store_scatter(idx2d, [rowv, colv + 16], f1)
        return jnp.int32(0)

    lax.fori_loop(0, _NV, gen_body, jnp.int32(0))

    def _chunk_copy(c):
        return pltpu.make_async_copy(
            zeros_v, out_hbm.at[plsc.Indices(idx2d.at[c], ignored_value=-1)],
            sem)

    def fire_body(c, _):
        _chunk_copy(c).start()

        @pl.when(c >= _INFLIGHT)
        def _():
            _chunk_copy(c - _INFLIGHT).wait()

        return jnp.int32(0)

    lax.fori_loop(0, _NCH_MAX, fire_body, jnp.int32(0))

    def drain_body(c, _):
        _chunk_copy(c).wait()
        return jnp.int32(0)

    lax.fori_loop(_NCH_MAX - _INFLIGHT, _NCH_MAX, drain_body, jnp.int32(0))


@functools.cache
def _sc_scatter():
    mesh = plsc.VectorSubcoreMesh(core_axis_name="c", subcore_axis_name="s")
    return pl.kernel(
        _sc_body,
        out_type=(),
        mesh=mesh,
        compiler_params=pltpu.CompilerParams(needs_layout_passes=False),
        scratch_types=[
            pltpu.VMEM((_G,), jnp.int32),
            pltpu.VMEM((_G,), jnp.int32),
            pltpu.VMEM((_G,), jnp.int32),
            pltpu.VMEM((_G,), jnp.int32),
            pltpu.VMEM((_G,), jnp.float32),
            pltpu.VMEM((_G,), jnp.float32),
            pltpu.VMEM((_NCH_MAX, _CHUNK), jnp.int32),
            pltpu.VMEM((_CHUNK,), jnp.float32),
            pltpu.SemaphoreType.DMA,
        ],
    )


def kernel(x, gens, pd):
    npad = _PPAD - _P
    zpad = jnp.zeros((npad,), jnp.int32)
    cols = [jnp.concatenate([gens[:, j], zpad]) for j in range(4)]
    pd_b = jnp.concatenate([pd[:, 0], jnp.zeros((npad,), jnp.float32)])
    pd_d = jnp.concatenate([pd[:, 1], jnp.ones((npad,), jnp.float32)])

    out = _tc_copy(x).reshape(_HW)
    out_ref = jax.new_ref(out)
    _sc_scatter()(out_ref, *cols, pd_b, pd_d)
    return out_ref[...].reshape(_H, _W)


# speedup vs baseline: 1.0097x; 1.0097x over previous
import functools

import jax
import jax.numpy as jnp
from jax import lax
from jax.experimental import pallas as pl
from jax.experimental.pallas import tpu as pltpu
from jax.experimental.pallas import tpu_sc as plsc

_H = 4096
_W = 4096
_HW = _H * _W
_P = 500000
_THETA = 0.5

_NC = 2
_NS = 16
_NW = _NC * _NS

_PPAD = 503808
_G = _PPAD // _NW
_NB = _G // 128
_CHUNK = 128
_NCH = 2 * _NB
_INFLIGHT = 32


def _copy_body(x_ref, o_ref):
    o_ref[...] = x_ref[...]


_tc_copy = pl.pallas_call(
    _copy_body,
    out_shape=jax.ShapeDtypeStruct((_H, _W), jnp.float32),
    grid=(32,),
    in_specs=[pl.BlockSpec((_H // 32, _W), lambda i: (i, 0))],
    out_specs=pl.BlockSpec((_H // 32, _W), lambda i: (i, 0)),
)


def _sc_body(out_hbm, r0_h, c0_h, r1_h, c1_h, b_h, d_h,
             r0_v, c0_v, r1_v, c1_v, b_v, d_v, idx2d, zeros_v, sem):
    wid = lax.axis_index("s") * _NC + lax.axis_index("c")
    base = wid * _G

    stage = [(r0_h, r0_v), (c0_h, c0_v), (r1_h, r1_v),
             (c1_h, c1_v), (b_h, b_v), (d_h, d_v)]
    for src, dst in stage:
        pltpu.async_copy(src.at[pl.ds(base, _G)], dst, sem)
    for src, dst in stage:
        pltpu.make_async_copy(src.at[pl.ds(base, _G)], dst, sem).wait()

    for j in range(_CHUNK // 16):
        zeros_v[pl.ds(j * 16, 16)] = jnp.zeros((16,), jnp.float32)

    def gen_body(blk, _):
        for j in range(8):
            sl = pl.ds((blk * 8 + j) * 16, 16)
            m = jnp.abs(d_v[sl] - b_v[sl]) <= _THETA
            f0 = jnp.where(m, r0_v[sl] * _W + c0_v[sl], -1)
            f1 = jnp.where(m, r1_v[sl] * _W + c1_v[sl], -1)
            idx2d[blk, pl.ds(j * 16, 16)] = f0
            idx2d[blk + _NB, pl.ds(j * 16, 16)] = f1
        return jnp.int32(0)

    lax.fori_loop(0, _NB, gen_body, jnp.int32(0))

    def _chunk_copy(c):
        return pltpu.make_async_copy(
            zeros_v, out_hbm.at[plsc.Indices(idx2d.at[c], ignored_value=-1)],
            sem)

    def fire_body(c, _):
        _chunk_copy(c).start()

        @pl.when(c >= _INFLIGHT)
        def _():
            _chunk_copy(c - _INFLIGHT).wait()

        return jnp.int32(0)

    lax.fori_loop(0, _NCH, fire_body, jnp.int32(0))

    def drain_body(c, _):
        _chunk_copy(c).wait()
        return jnp.int32(0)

    lax.fori_loop(_NCH - _INFLIGHT, _NCH, drain_body, jnp.int32(0))


@functools.cache
def _sc_scatter():
    mesh = plsc.VectorSubcoreMesh(core_axis_name="c", subcore_axis_name="s")
    return pl.kernel(
        _sc_body,
        out_type=(),
        mesh=mesh,
        compiler_params=pltpu.CompilerParams(needs_layout_passes=False),
        scratch_types=[
            pltpu.VMEM((_G,), jnp.int32),
            pltpu.VMEM((_G,), jnp.int32),
            pltpu.VMEM((_G,), jnp.int32),
            pltpu.VMEM((_G,), jnp.int32),
            pltpu.VMEM((_G,), jnp.float32),
            pltpu.VMEM((_G,), jnp.float32),
            pltpu.VMEM((_NCH, _CHUNK), jnp.int32),
            pltpu.VMEM((_CHUNK,), jnp.float32),
            pltpu.SemaphoreType.DMA,
        ],
    )


def kernel(x, gens, pd):
    npad = _PPAD - _P
    zpad = jnp.zeros((npad,), jnp.int32)
    cols = [jnp.concatenate([gens[:, j], zpad]) for j in range(4)]
    pd_b = jnp.concatenate([pd[:, 0], jnp.zeros((npad,), jnp.float32)])
    pd_d = jnp.concatenate([pd[:, 1], jnp.ones((npad,), jnp.float32)])

    out = _tc_copy(x).reshape(_HW)
    out_ref = jax.new_ref(out)
    _sc_scatter()(out_ref, *cols, pd_b, pd_d)
    return out_ref[...].reshape(_H, _W)
